# Initial kernel scaffold; baseline (speedup 1.0000x reference)
import jax
import jax.numpy as jnp
from jax.experimental import pallas as pl


def kernel(x, W_conv, b_conv, W_cls, b_cls, W_reg, b_reg):
    raise NotImplementedError("write your pallas kernel here")


# speedup vs baseline: 1.1923x; 1.1923x over previous
import jax
import jax.numpy as jnp
from jax import lax
from jax.experimental import pallas as pl

_TH = 16
_H = 384
_W = 384
_WP = _W + 2
_CI = 96
_CO_CLS = 24
_CO_REG = 48
_CO = _CO_CLS + _CO_REG


def _rpn_body(body_ref, top_ref, bot_ref, wk_ref, wcr_ref, bcv_ref, bcr_ref,
              out_ref):
    i = pl.program_id(0)
    nt = pl.num_programs(0)
    top = jnp.where(i == 0, 0.0, top_ref[...])
    bot = jnp.where(i == nt - 1, 0.0, bot_ref[...])
    xt = jnp.concatenate([top, body_ref[...], bot], axis=0)

    acc = jnp.zeros((_TH * _W, _CI), jnp.float32)
    for dx in range(3):
        xs = xt[:, dx:dx + _W, :]
        for dy in range(3):
            a = xs[dy:dy + _TH].reshape(_TH * _W, _CI)
            w = wk_ref[dy * 3 + dx]
            acc += lax.dot_general(a, w, (((1,), (0,)), ((), ())),
                                   preferred_element_type=jnp.float32)
    h = jnp.maximum(acc + bcv_ref[...], 0.0)
    o = lax.dot_general(h, wcr_ref[...], (((1,), (0,)), ((), ())),
                        preferred_element_type=jnp.float32) + bcr_ref[...]
    out_ref[...] = o.reshape(_TH, _W, _CO)


def kernel(x, W_conv, b_conv, W_cls, b_cls, W_reg, b_reg):
    xh = jnp.pad(x[0].transpose(1, 2, 0), ((0, 0), (1, 1), (0, 0)))
    wk = W_conv.transpose(2, 3, 1, 0).reshape(9, _CI, _CI)
    wcr = jnp.concatenate([W_cls[:, :, 0, 0].T, W_reg[:, :, 0, 0].T], axis=1)
    bcv = b_conv.reshape(1, _CI)
    bcr = jnp.concatenate([b_cls, b_reg]).reshape(1, _CO)

    nt = _H // _TH
    out_hwc = pl.pallas_call(
        _rpn_body,
        grid=(nt,),
        in_specs=[
            pl.BlockSpec((_TH, _WP, _CI), lambda i: (i, 0, 0)),
            pl.BlockSpec((1, _WP, _CI),
                         lambda i: (jnp.maximum(i * _TH - 1, 0), 0, 0)),
            pl.BlockSpec((1, _WP, _CI),
                         lambda i: (jnp.minimum(i * _TH + _TH, _H - 1), 0, 0)),
            pl.BlockSpec((9, _CI, _CI), lambda i: (0, 0, 0)),
            pl.BlockSpec((_CI, _CO), lambda i: (0, 0)),
            pl.BlockSpec((1, _CI), lambda i: (0, 0)),
            pl.BlockSpec((1, _CO), lambda i: (0, 0)),
        ],
        out_specs=pl.BlockSpec((_TH, _W, _CO), lambda i: (i, 0, 0)),
        out_shape=jax.ShapeDtypeStruct((_H, _W, _CO), jnp.float32),
    )(xh, xh, xh, wk, wcr, bcv, bcr)
    o = out_hwc.transpose(2, 0, 1)
    cls_out = o[:_CO_CLS][None]
    reg_out = o[_CO_CLS:][None]
    return (cls_out, reg_out)


# speedup vs baseline: 1.2544x; 1.0520x over previous
import jax
import jax.numpy as jnp
from jax import lax
from jax.experimental import pallas as pl

_TH = 16
_H = 384
_W = 384
_WP = _W + 2
_CI = 96
_CO_CLS = 24
_CO_REG = 48
_CO = _CO_CLS + _CO_REG


def _rpn_body(body_ref, top_ref, bot_ref, wk_ref, wcr_ref, bcv_ref, bcr_ref,
              out_ref):
    i = pl.program_id(0)
    nt = pl.num_programs(0)
    top = jnp.where(i == 0, 0.0, top_ref[...])
    bot = jnp.where(i == nt - 1, 0.0, bot_ref[...])
    xt = jnp.concatenate([top, body_ref[...], bot], axis=0)

    acc = jnp.zeros((_TH * _W, _CI), jnp.float32)
    for dx in range(3):
        xs = xt[:, dx:dx + _W, :]
        for dy in range(3):
            a = xs[dy:dy + _TH].reshape(_TH * _W, _CI)
            w = wk_ref[dy * 3 + dx]
            acc += lax.dot_general(a, w, (((1,), (0,)), ((), ())),
                                   preferred_element_type=jnp.float32)
    h = jnp.maximum(acc + bcv_ref[...], 0.0).astype(jnp.bfloat16)
    o = lax.dot_general(h, wcr_ref[...], (((1,), (0,)), ((), ())),
                        preferred_element_type=jnp.float32) + bcr_ref[...]
    out_ref[...] = o.reshape(_TH, _W, _CO)


def kernel(x, W_conv, b_conv, W_cls, b_cls, W_reg, b_reg):
    xh = jnp.pad(x[0].transpose(1, 2, 0), ((0, 0), (1, 1), (0, 0)))
    xh = xh.astype(jnp.bfloat16)
    wk = W_conv.transpose(2, 3, 1, 0).reshape(9, _CI, _CI).astype(jnp.bfloat16)
    wcr = jnp.concatenate([W_cls[:, :, 0, 0].T, W_reg[:, :, 0, 0].T],
                          axis=1).astype(jnp.bfloat16)
    bcv = b_conv.reshape(1, _CI)
    bcr = jnp.concatenate([b_cls, b_reg]).reshape(1, _CO)

    nt = _H // _TH
    out_hwc = pl.pallas_call(
        _rpn_body,
        grid=(nt,),
        in_specs=[
            pl.BlockSpec((_TH, _WP, _CI), lambda i: (i, 0, 0)),
            pl.BlockSpec((1, _WP, _CI),
                         lambda i: (jnp.maximum(i * _TH - 1, 0), 0, 0)),
            pl.BlockSpec((1, _WP, _CI),
                         lambda i: (jnp.minimum(i * _TH + _TH, _H - 1), 0, 0)),
            pl.BlockSpec((9, _CI, _CI), lambda i: (0, 0, 0)),
            pl.BlockSpec((_CI, _CO), lambda i: (0, 0)),
            pl.BlockSpec((1, _CI), lambda i: (0, 0)),
            pl.BlockSpec((1, _CO), lambda i: (0, 0)),
        ],
        out_specs=pl.BlockSpec((_TH, _W, _CO), lambda i: (i, 0, 0)),
        out_shape=jax.ShapeDtypeStruct((_H, _W, _CO), jnp.float32),
    )(xh, xh, xh, wk, wcr, bcv, bcr)
    o = out_hwc.transpose(2, 0, 1)
    cls_out = o[:_CO_CLS][None]
    reg_out = o[_CO_CLS:][None]
    return (cls_out, reg_out)


# speedup vs baseline: 1.9765x; 1.5757x over previous
import jax
import jax.numpy as jnp
from jax import lax
from jax.experimental import pallas as pl

_TH = 16
_H = 384
_W = 384
_CI = 96
_CO_CLS = 24
_CO_REG = 48
_CO = _CO_CLS + _CO_REG
_N = _TH * _W


def _dot(a, b):
    return lax.dot_general(a, b, (((1,), (0,)), ((), ())),
                           preferred_element_type=jnp.float32)


def _rpn_body(body_ref, top_ref, bot_ref, wk_ref, wcr_ref, bcv_ref, bcr_ref,
              cls_ref, reg_ref):
    i = pl.program_id(0)
    nt = pl.num_programs(0)
    bodyt = body_ref[...].astype(jnp.bfloat16).T.reshape(_TH, _W, _CI)
    top = jnp.where(i == 0, jnp.bfloat16(0),
                    top_ref[...].astype(jnp.bfloat16)).T.reshape(1, _W, _CI)
    bot = jnp.where(i == nt - 1, jnp.bfloat16(0),
                    bot_ref[...].astype(jnp.bfloat16)).T.reshape(1, _W, _CI)
    xt = jnp.concatenate([top, bodyt, bot], axis=0)

    zcol = jnp.zeros((_TH + 2, 1, _CI), jnp.bfloat16)
    acc = jnp.zeros((_N, _CI), jnp.float32)
    for dx in range(3):
        if dx == 0:
            xs = jnp.concatenate([zcol, xt[:, :_W - 1, :]], axis=1)
        elif dx == 1:
            xs = xt
        else:
            xs = jnp.concatenate([xt[:, 1:, :], zcol], axis=1)
        for dy in range(3):
            a = xs[dy:dy + _TH].reshape(_N, _CI)
            acc += _dot(a, wk_ref[dy * 3 + dx])
    h = jnp.maximum(acc + bcv_ref[...], 0.0).astype(jnp.bfloat16)
    o = _dot(h, wcr_ref[...]) + bcr_ref[...]
    ot = o.T
    cls_ref[...] = ot[:_CO_CLS]
    reg_ref[...] = ot[_CO_CLS:]


def kernel(x, W_conv, b_conv, W_cls, b_cls, W_reg, b_reg):
    xin = x[0].reshape(_CI, _H * _W)
    wk = W_conv.transpose(2, 3, 1, 0).reshape(9, _CI, _CI).astype(jnp.bfloat16)
    wcr = jnp.concatenate([W_cls[:, :, 0, 0].T, W_reg[:, :, 0, 0].T],
                          axis=1).astype(jnp.bfloat16)
    bcv = b_conv.reshape(1, _CI)
    bcr = jnp.concatenate([b_cls, b_reg]).reshape(1, _CO)

    nt = _H // _TH
    cls2d, reg2d = pl.pallas_call(
        _rpn_body,
        grid=(nt,),
        in_specs=[
            pl.BlockSpec((_CI, _N), lambda i: (0, i)),
            pl.BlockSpec((_CI, _W), lambda i: (0, jnp.maximum(i * _TH - 1, 0))),
            pl.BlockSpec((_CI, _W),
                         lambda i: (0, jnp.minimum(i * _TH + _TH, _H - 1))),
            pl.BlockSpec((9, _CI, _CI), lambda i: (0, 0, 0)),
            pl.BlockSpec((_CI, _CO), lambda i: (0, 0)),
            pl.BlockSpec((1, _CI), lambda i: (0, 0)),
            pl.BlockSpec((1, _CO), lambda i: (0, 0)),
        ],
        out_specs=[
            pl.BlockSpec((_CO_CLS, _N), lambda i: (0, i)),
            pl.BlockSpec((_CO_REG, _N), lambda i: (0, i)),
        ],
        out_shape=[
            jax.ShapeDtypeStruct((_CO_CLS, _H * _W), jnp.float32),
            jax.ShapeDtypeStruct((_CO_REG, _H * _W), jnp.float32),
        ],
    )(xin, xin, xin, wk, wcr, bcv, bcr)
    cls_out = cls2d.reshape(1, _CO_CLS, _H, _W)
    reg_out = reg2d.reshape(1, _CO_REG, _H, _W)
    return (cls_out, reg_out)


# speedup vs baseline: 2.0404x; 1.0323x over previous
import jax
import jax.numpy as jnp
from jax import lax
from jax.experimental import pallas as pl

_TH = 16
_H = 384
_W = 384
_CI = 96
_CO_CLS = 24
_CO_REG = 48
_CO = _CO_CLS + _CO_REG
_N = _TH * _W


def _dot(a, b):
    return lax.dot_general(a, b, (((1,), (0,)), ((), ())),
                           preferred_element_type=jnp.float32)


def _rpn_body(body_ref, top_ref, bot_ref, wk_ref, wcr_ref, bcv_ref, bcr_ref,
              cls_ref, reg_ref):
    i = pl.program_id(0)
    nt = pl.num_programs(0)
    bodyt = body_ref[...].astype(jnp.bfloat16).T.reshape(_TH, _W, _CI)
    top = jnp.where(i == 0, jnp.bfloat16(0),
                    top_ref[...].astype(jnp.bfloat16)).T.reshape(1, _W, _CI)
    bot = jnp.where(i == nt - 1, jnp.bfloat16(0),
                    bot_ref[...].astype(jnp.bfloat16)).T.reshape(1, _W, _CI)
    xt = jnp.concatenate([top, bodyt, bot], axis=0)

    zcol = jnp.zeros((_TH + 2, 1, _CI), jnp.bfloat16)
    taps = []
    for dx in range(3):
        if dx == 0:
            xs = jnp.concatenate([zcol, xt[:, :_W - 1, :]], axis=1)
        elif dx == 1:
            xs = xt
        else:
            xs = jnp.concatenate([xt[:, 1:, :], zcol], axis=1)
        for dy in range(3):
            taps.append(xs[dy:dy + _TH].reshape(_N, _CI))
    a = jnp.concatenate(taps, axis=1)
    acc = _dot(a, wk_ref[...])
    h = jnp.maximum(acc + bcv_ref[...], 0.0).astype(jnp.bfloat16)
    o = _dot(h, wcr_ref[...]) + bcr_ref[...]
    ot = o.T
    cls_ref[...] = ot[:_CO_CLS]
    reg_ref[...] = ot[_CO_CLS:]


def kernel(x, W_conv, b_conv, W_cls, b_cls, W_reg, b_reg):
    xin = x[0].reshape(_CI, _H * _W)
    wk = W_conv.transpose(3, 2, 1, 0).reshape(9 * _CI, _CI).astype(jnp.bfloat16)
    wcr = jnp.concatenate([W_cls[:, :, 0, 0].T, W_reg[:, :, 0, 0].T],
                          axis=1).astype(jnp.bfloat16)
    bcv = b_conv.reshape(1, _CI)
    bcr = jnp.concatenate([b_cls, b_reg]).reshape(1, _CO)

    nt = _H // _TH
    cls2d, reg2d = pl.pallas_call(
        _rpn_body,
        grid=(nt,),
        in_specs=[
            pl.BlockSpec((_CI, _N), lambda i: (0, i)),
            pl.BlockSpec((_CI, _W), lambda i: (0, jnp.maximum(i * _TH - 1, 0))),
            pl.BlockSpec((_CI, _W),
                         lambda i: (0, jnp.minimum(i * _TH + _TH, _H - 1))),
            pl.BlockSpec((9 * _CI, _CI), lambda i: (0, 0)),
            pl.BlockSpec((_CI, _CO), lambda i: (0, 0)),
            pl.BlockSpec((1, _CI), lambda i: (0, 0)),
            pl.BlockSpec((1, _CO), lambda i: (0, 0)),
        ],
        out_specs=[
            pl.BlockSpec((_CO_CLS, _N), lambda i: (0, i)),
            pl.BlockSpec((_CO_REG, _N), lambda i: (0, i)),
        ],
        out_shape=[
            jax.ShapeDtypeStruct((_CO_CLS, _H * _W), jnp.float32),
            jax.ShapeDtypeStruct((_CO_REG, _H * _W), jnp.float32),
        ],
    )(xin, xin, xin, wk, wcr, bcv, bcr)
    cls_out = cls2d.reshape(1, _CO_CLS, _H, _W)
    reg_out = reg2d.reshape(1, _CO_REG, _H, _W)
    return (cls_out, reg_out)


# speedup vs baseline: 3.4590x; 1.6953x over previous
import jax
import jax.numpy as jnp
from jax import lax
from jax.experimental import pallas as pl

_TH = 16
_H = 384
_W = 384
_CI = 96
_CO_CLS = 24
_CO_REG = 48
_CO = _CO_CLS + _CO_REG
_N = _TH * _W


def _dot(a, b):
    return lax.dot_general(a, b, (((1,), (0,)), ((), ())),
                           preferred_element_type=jnp.float32)


def _rpn_body(body_ref, top_ref, bot_ref, wk_ref, wcr_ref, bcv_ref, bcr_ref,
              cls_ref, reg_ref):
    i = pl.program_id(0)
    nt = pl.num_programs(0)
    body2d = body_ref[...].reshape(_CI, _N)
    bodyt = body2d.astype(jnp.bfloat16).T.reshape(_TH, _W, _CI)
    topt = top_ref[...].reshape(_CI, 8 * _W).astype(jnp.bfloat16).T
    top = jnp.where(i == 0, jnp.bfloat16(0),
                    topt.reshape(8, _W, _CI)[7:8])
    bott = bot_ref[...].reshape(_CI, 8 * _W).astype(jnp.bfloat16).T
    bot = jnp.where(i == nt - 1, jnp.bfloat16(0),
                    bott.reshape(8, _W, _CI)[0:1])
    xt = jnp.concatenate([top, bodyt, bot], axis=0)

    zcol = jnp.zeros((_TH + 2, 1, _CI), jnp.bfloat16)
    taps = []
    for dx in range(3):
        if dx == 0:
            xs = jnp.concatenate([zcol, xt[:, :_W - 1, :]], axis=1)
        elif dx == 1:
            xs = xt
        else:
            xs = jnp.concatenate([xt[:, 1:, :], zcol], axis=1)
        for dy in range(3):
            taps.append(xs[dy:dy + _TH].reshape(_N, _CI))
    a = jnp.concatenate(taps, axis=1)
    acc = _dot(a, wk_ref[...])
    h = jnp.maximum(acc + bcv_ref[...], 0.0).astype(jnp.bfloat16)
    o = _dot(h, wcr_ref[...]) + bcr_ref[...]
    ot = o.T
    cls_ref[...] = ot[:_CO_CLS].reshape(_CO_CLS, _TH, _W)
    reg_ref[...] = ot[_CO_CLS:].reshape(_CO_REG, _TH, _W)


def kernel(x, W_conv, b_conv, W_cls, b_cls, W_reg, b_reg):
    xin = x[0]
    wk = W_conv.transpose(3, 2, 1, 0).reshape(9 * _CI, _CI).astype(jnp.bfloat16)
    wcr = jnp.concatenate([W_cls[:, :, 0, 0].T, W_reg[:, :, 0, 0].T],
                          axis=1).astype(jnp.bfloat16)
    bcv = b_conv.reshape(1, _CI)
    bcr = jnp.concatenate([b_cls, b_reg]).reshape(1, _CO)

    nt = _H // _TH
    cls2d, reg2d = pl.pallas_call(
        _rpn_body,
        grid=(nt,),
        in_specs=[
            pl.BlockSpec((_CI, _TH, _W), lambda i: (0, i, 0)),
            pl.BlockSpec((_CI, 8, _W),
                         lambda i: (0, jnp.maximum((i * _TH - 1) // 8, 0), 0)),
            pl.BlockSpec((_CI, 8, _W),
                         lambda i: (0, jnp.minimum((i * _TH + _TH) // 8,
                                                   _H // 8 - 1), 0)),
            pl.BlockSpec((9 * _CI, _CI), lambda i: (0, 0)),
            pl.BlockSpec((_CI, _CO), lambda i: (0, 0)),
            pl.BlockSpec((1, _CI), lambda i: (0, 0)),
            pl.BlockSpec((1, _CO), lambda i: (0, 0)),
        ],
        out_specs=[
            pl.BlockSpec((_CO_CLS, _TH, _W), lambda i: (0, i, 0)),
            pl.BlockSpec((_CO_REG, _TH, _W), lambda i: (0, i, 0)),
        ],
        out_shape=[
            jax.ShapeDtypeStruct((_CO_CLS, _H, _W), jnp.float32),
            jax.ShapeDtypeStruct((_CO_REG, _H, _W), jnp.float32),
        ],
    )(xin, xin, xin, wk, wcr, bcv, bcr)
    return (cls2d[None], reg2d[None])


# speedup vs baseline: 3.6804x; 1.0640x over previous
import jax
import jax.numpy as jnp
from jax import lax
from jax.experimental import pallas as pl

_TH = 24
_H = 384
_W = 384
_CI = 96
_CO_CLS = 24
_CO_REG = 48
_CO = _CO_CLS + _CO_REG
_N = _TH * _W
_KSPLIT = 5


def _dot(a, b):
    return lax.dot_general(a, b, (((1,), (0,)), ((), ())),
                           preferred_element_type=jnp.float32)


def _rpn_body(body_ref, top_ref, bot_ref, wk_ref, wcr_ref, bcv_ref, bcr_ref,
              cls_ref, reg_ref):
    i = pl.program_id(0)
    nt = pl.num_programs(0)
    body2d = body_ref[...].reshape(_CI, _N)
    bodyt = body2d.astype(jnp.bfloat16).T.reshape(_TH, _W, _CI)
    topt = top_ref[...].reshape(_CI, 8 * _W).astype(jnp.bfloat16).T
    top = jnp.where(i == 0, jnp.bfloat16(0),
                    topt.reshape(8, _W, _CI)[7:8])
    bott = bot_ref[...].reshape(_CI, 8 * _W).astype(jnp.bfloat16).T
    bot = jnp.where(i == nt - 1, jnp.bfloat16(0),
                    bott.reshape(8, _W, _CI)[0:1])
    xt = jnp.concatenate([top, bodyt, bot], axis=0)

    zcol = jnp.zeros((_TH + 2, 1, _CI), jnp.bfloat16)
    taps = []
    for dx in range(3):
        if dx == 0:
            xs = jnp.concatenate([zcol, xt[:, :_W - 1, :]], axis=1)
        elif dx == 1:
            xs = xt
        else:
            xs = jnp.concatenate([xt[:, 1:, :], zcol], axis=1)
        for dy in range(3):
            taps.append(xs[dy:dy + _TH].reshape(_N, _CI))
    a = jnp.concatenate(taps, axis=1)
    acc = _dot(a, wk_ref[...])
    h = jnp.maximum(acc + bcv_ref[...], 0.0).astype(jnp.bfloat16)
    o = _dot(h, wcr_ref[...]) + bcr_ref[...]
    ot = o.T
    cls_ref[...] = ot[:_CO_CLS].reshape(_CO_CLS, _TH, _W)
    reg_ref[...] = ot[_CO_CLS:].reshape(_CO_REG, _TH, _W)


def kernel(x, W_conv, b_conv, W_cls, b_cls, W_reg, b_reg):
    xin = x[0]
    wk = W_conv.transpose(3, 2, 1, 0).reshape(9 * _CI, _CI).astype(jnp.bfloat16)
    wcr = jnp.concatenate([W_cls[:, :, 0, 0].T, W_reg[:, :, 0, 0].T],
                          axis=1).astype(jnp.bfloat16)
    bcv = b_conv.reshape(1, _CI)
    bcr = jnp.concatenate([b_cls, b_reg]).reshape(1, _CO)

    nt = _H // _TH
    cls3d, reg3d = pl.pallas_call(
        _rpn_body,
        grid=(nt,),
        in_specs=[
            pl.BlockSpec((_CI, _TH, _W), lambda i: (0, i, 0)),
            pl.BlockSpec((_CI, 8, _W),
                         lambda i: (0, jnp.maximum((i * _TH - 1) // 8, 0), 0)),
            pl.BlockSpec((_CI, 8, _W),
                         lambda i: (0, jnp.minimum((i * _TH + _TH) // 8,
                                                   _H // 8 - 1), 0)),
            pl.BlockSpec((9 * _CI, _CI), lambda i: (0, 0)),
            pl.BlockSpec((_CI, _CO), lambda i: (0, 0)),
            pl.BlockSpec((1, _CI), lambda i: (0, 0)),
            pl.BlockSpec((1, _CO), lambda i: (0, 0)),
        ],
        out_specs=[
            pl.BlockSpec((_CO_CLS, _TH, _W), lambda i: (0, i, 0)),
            pl.BlockSpec((_CO_REG, _TH, _W), lambda i: (0, i, 0)),
        ],
        out_shape=[
            jax.ShapeDtypeStruct((_CO_CLS, _H, _W), jnp.float32),
            jax.ShapeDtypeStruct((_CO_REG, _H, _W), jnp.float32),
        ],
    )(xin, xin, xin, wk, wcr, bcv, bcr)
    return (cls3d[None], reg3d[None])


# speedup vs baseline: 3.6817x; 1.0003x over previous
import jax
import jax.numpy as jnp
from jax import lax
from jax.experimental import pallas as pl

_TH = 24
_H = 384
_W = 384
_CI = 96
_CO_CLS = 24
_CO_REG = 48
_CO = _CO_CLS + _CO_REG
_N = _TH * _W
_KSPLIT = 5


def _dot(a, b):
    return lax.dot_general(a, b, (((1,), (0,)), ((), ())),
                           preferred_element_type=jnp.float32)


def _rpn_body(body_ref, top_ref, bot_ref, wk_ref, wcr_ref, bcv_ref, bcr_ref,
              cls_ref, reg_ref):
    i = pl.program_id(0)
    nt = pl.num_programs(0)
    body2d = body_ref[...].astype(jnp.bfloat16).reshape(_CI, _N)
    bodyt = body2d.T.reshape(_TH, _W, _CI)
    topt = top_ref[...].astype(jnp.bfloat16).reshape(_CI, 8 * _W).T
    top = jnp.where(i == 0, jnp.bfloat16(0),
                    topt.reshape(8, _W, _CI)[7:8])
    bott = bot_ref[...].astype(jnp.bfloat16).reshape(_CI, 8 * _W).T
    bot = jnp.where(i == nt - 1, jnp.bfloat16(0),
                    bott.reshape(8, _W, _CI)[0:1])
    xt = jnp.concatenate([top, bodyt, bot], axis=0)

    zcol = jnp.zeros((_TH + 2, 1, _CI), jnp.bfloat16)
    taps = []
    for dx in range(3):
        if dx == 0:
            xs = jnp.concatenate([zcol, xt[:, :_W - 1, :]], axis=1)
        elif dx == 1:
            xs = xt
        else:
            xs = jnp.concatenate([xt[:, 1:, :], zcol], axis=1)
        for dy in range(3):
            taps.append(xs[dy:dy + _TH].reshape(_N, _CI))
    a = jnp.concatenate(taps, axis=1)
    acc = _dot(a, wk_ref[...])
    h = jnp.maximum(acc + bcv_ref[...], 0.0).astype(jnp.bfloat16)
    o = _dot(h, wcr_ref[...]) + bcr_ref[...]
    ot = o.T
    cls_ref[...] = ot[:_CO_CLS].reshape(_CO_CLS, _TH, _W)
    reg_ref[...] = ot[_CO_CLS:].reshape(_CO_REG, _TH, _W)


def kernel(x, W_conv, b_conv, W_cls, b_cls, W_reg, b_reg):
    xin = x[0]
    wk = W_conv.transpose(3, 2, 1, 0).reshape(9 * _CI, _CI).astype(jnp.bfloat16)
    wcr = jnp.concatenate([W_cls[:, :, 0, 0].T, W_reg[:, :, 0, 0].T],
                          axis=1).astype(jnp.bfloat16)
    bcv = b_conv.reshape(1, _CI)
    bcr = jnp.concatenate([b_cls, b_reg]).reshape(1, _CO)

    nt = _H // _TH
    cls3d, reg3d = pl.pallas_call(
        _rpn_body,
        grid=(nt,),
        in_specs=[
            pl.BlockSpec((_CI, _TH, _W), lambda i: (0, i, 0)),
            pl.BlockSpec((_CI, 8, _W),
                         lambda i: (0, jnp.maximum((i * _TH - 1) // 8, 0), 0)),
            pl.BlockSpec((_CI, 8, _W),
                         lambda i: (0, jnp.minimum((i * _TH + _TH) // 8,
                                                   _H // 8 - 1), 0)),
            pl.BlockSpec((9 * _CI, _CI), lambda i: (0, 0)),
            pl.BlockSpec((_CI, _CO), lambda i: (0, 0)),
            pl.BlockSpec((1, _CI), lambda i: (0, 0)),
            pl.BlockSpec((1, _CO), lambda i: (0, 0)),
        ],
        out_specs=[
            pl.BlockSpec((_CO_CLS, _TH, _W), lambda i: (0, i, 0)),
            pl.BlockSpec((_CO_REG, _TH, _W), lambda i: (0, i, 0)),
        ],
        out_shape=[
            jax.ShapeDtypeStruct((_CO_CLS, _H, _W), jnp.float32),
            jax.ShapeDtypeStruct((_CO_REG, _H, _W), jnp.float32),
        ],
    )(xin, xin, xin, wk, wcr, bcv, bcr)
    return (cls3d[None], reg3d[None])


# speedup vs baseline: 3.6827x; 1.0003x over previous
import jax
import jax.numpy as jnp
from jax import lax
from jax.experimental import pallas as pl
from jax.experimental.pallas import tpu as pltpu

_TH = 24
_H = 384
_W = 384
_CI = 96
_CO_CLS = 24
_CO_REG = 48
_CO = _CO_CLS + _CO_REG
_N = _TH * _W
_KSPLIT = 5


def _dot(a, b):
    return lax.dot_general(a, b, (((1,), (0,)), ((), ())),
                           preferred_element_type=jnp.float32)


def _rpn_body(body_ref, top_ref, bot_ref, wk_ref, wcr_ref, bcv_ref, bcr_ref,
              cls_ref, reg_ref):
    i = pl.program_id(0)
    nt = pl.num_programs(0)
    body2d = body_ref[...].astype(jnp.bfloat16).reshape(_CI, _N)
    bodyt = body2d.T.reshape(_TH, _W, _CI)
    topt = top_ref[...].astype(jnp.bfloat16).reshape(_CI, 8 * _W).T
    top = jnp.where(i == 0, jnp.bfloat16(0),
                    topt.reshape(8, _W, _CI)[7:8])
    bott = bot_ref[...].astype(jnp.bfloat16).reshape(_CI, 8 * _W).T
    bot = jnp.where(i == nt - 1, jnp.bfloat16(0),
                    bott.reshape(8, _W, _CI)[0:1])
    xt = jnp.concatenate([top, bodyt, bot], axis=0)

    zcol = jnp.zeros((_TH + 2, 1, _CI), jnp.bfloat16)
    taps = []
    for dx in range(3):
        if dx == 0:
            xs = jnp.concatenate([zcol, xt[:, :_W - 1, :]], axis=1)
        elif dx == 1:
            xs = xt
        else:
            xs = jnp.concatenate([xt[:, 1:, :], zcol], axis=1)
        for dy in range(3):
            taps.append(xs[dy:dy + _TH].reshape(_N, _CI))
    a = jnp.concatenate(taps, axis=1)
    acc = _dot(a, wk_ref[...])
    h = jnp.maximum(acc + bcv_ref[...], 0.0).astype(jnp.bfloat16)
    o = _dot(h, wcr_ref[...]) + bcr_ref[...]
    ot = o.T
    cls_ref[...] = ot[:_CO_CLS].reshape(_CO_CLS, _TH, _W)
    reg_ref[...] = ot[_CO_CLS:].reshape(_CO_REG, _TH, _W)


def kernel(x, W_conv, b_conv, W_cls, b_cls, W_reg, b_reg):
    xin = x[0]
    wk = W_conv.transpose(3, 2, 1, 0).reshape(9 * _CI, _CI).astype(jnp.bfloat16)
    wcr = jnp.concatenate([W_cls[:, :, 0, 0].T, W_reg[:, :, 0, 0].T],
                          axis=1).astype(jnp.bfloat16)
    bcv = b_conv.reshape(1, _CI)
    bcr = jnp.concatenate([b_cls, b_reg]).reshape(1, _CO)

    nt = _H // _TH
    cls3d, reg3d = pl.pallas_call(
        _rpn_body,
        grid=(nt,),
        compiler_params=pltpu.CompilerParams(
            dimension_semantics=("parallel",)),
        in_specs=[
            pl.BlockSpec((_CI, _TH, _W), lambda i: (0, i, 0)),
            pl.BlockSpec((_CI, 8, _W),
                         lambda i: (0, jnp.maximum((i * _TH - 1) // 8, 0), 0)),
            pl.BlockSpec((_CI, 8, _W),
                         lambda i: (0, jnp.minimum((i * _TH + _TH) // 8,
                                                   _H // 8 - 1), 0)),
            pl.BlockSpec((9 * _CI, _CI), lambda i: (0, 0)),
            pl.BlockSpec((_CI, _CO), lambda i: (0, 0)),
            pl.BlockSpec((1, _CI), lambda i: (0, 0)),
            pl.BlockSpec((1, _CO), lambda i: (0, 0)),
        ],
        out_specs=[
            pl.BlockSpec((_CO_CLS, _TH, _W), lambda i: (0, i, 0)),
            pl.BlockSpec((_CO_REG, _TH, _W), lambda i: (0, i, 0)),
        ],
        out_shape=[
            jax.ShapeDtypeStruct((_CO_CLS, _H, _W), jnp.float32),
            jax.ShapeDtypeStruct((_CO_REG, _H, _W), jnp.float32),
        ],
    )(xin, xin, xin, wk, wcr, bcv, bcr)
    return (cls3d[None], reg3d[None])
